# baseline (device time: 33045 ns/iter reference)
import jax
import jax.numpy as jnp
from jax import lax
from jax.experimental import pallas as pl
from jax.experimental.pallas import tpu as pltpu

M = 1024
N = 1024


def kernel(x, w_mat):
    def body(x_ref, w_ref, out_ref, s1, r1, s2, r2, s3, r3, s4, r4,
             ssem, rsem):
        p = lax.axis_index("i")
        b0 = p & 1
        b1 = p >> 1
        n1 = p ^ 1
        n2 = 3 - p

        xb = x_ref[...].astype(jnp.bfloat16)
        wb = w_ref[...].astype(jnp.bfloat16)
        out_ref[...] = jnp.dot(xb, wb, preferred_element_type=jnp.float32)

        barrier = pltpu.get_barrier_semaphore()
        pl.semaphore_signal(barrier, inc=1, device_id=(n1,),
                            device_id_type=pl.DeviceIdType.MESH)
        pl.semaphore_signal(barrier, inc=1, device_id=(n2,),
                            device_id_type=pl.DeviceIdType.MESH)
        pl.semaphore_wait(barrier, 2)

        k1 = 256 * (((p + 1) >> 1) & 1)
        k2 = 512 + 256 * b1
        o1 = k1 + 128 * b1
        o2 = k2 + 128 * b0

        def exchange(phase, sbuf, rbuf, width, sends):
            rdmas = []
            for j, (partner, col) in enumerate(sends):
                sbuf[j, :, :] = out_ref[:, pl.ds(col, width)].astype(
                    jnp.bfloat16)
                rdma = pltpu.make_async_remote_copy(
                    src_ref=sbuf.at[j],
                    dst_ref=rbuf.at[j],
                    send_sem=ssem.at[2 * phase + j],
                    recv_sem=rsem.at[2 * phase + j],
                    device_id=(partner,),
                    device_id_type=pl.DeviceIdType.MESH,
                )
                rdma.start()
                rdmas.append(rdma)
            for rdma in rdmas:
                rdma.wait()

        def acc(col, width, rbuf, j):
            sl = pl.ds(col, width)
            out_ref[:, sl] = out_ref[:, sl] + rbuf[j, :, :].astype(
                jnp.float32)

        exchange(0, s1, r1, 256, [(n1, 256 - k1), (n2, 768 - 256 * b1)])
        acc(k1, 256, r1, 0)
        acc(k2, 256, r1, 1)

        exchange(1, s2, r2, 128,
                 [(n2, k1 + 128 - 128 * b1), (n1, k2 + 128 - 128 * b0)])
        acc(o1, 128, r2, 0)
        acc(o2, 128, r2, 1)

        exchange(2, s3, r3, 128, [(n2, o1), (n1, o2)])
        out_ref[:, pl.ds(k1 + 128 - 128 * b1, 128)] = r3[0, :, :].astype(
            jnp.float32)
        out_ref[:, pl.ds(k2 + 128 - 128 * b0, 128)] = r3[1, :, :].astype(
            jnp.float32)

        exchange(3, s4, r4, 256, [(n1, k1), (n2, k2)])
        out_ref[:, pl.ds(256 - k1, 256)] = r4[0, :, :].astype(jnp.float32)
        out_ref[:, pl.ds(768 - 256 * b1, 256)] = r4[1, :, :].astype(
            jnp.float32)

    return pl.pallas_call(
        body,
        out_shape=jax.ShapeDtypeStruct((M, N), jnp.float32),
        in_specs=[
            pl.BlockSpec(memory_space=pltpu.VMEM),
            pl.BlockSpec(memory_space=pltpu.VMEM),
        ],
        out_specs=pl.BlockSpec(memory_space=pltpu.VMEM),
        scratch_shapes=[
            pltpu.VMEM((2, M, 256), jnp.bfloat16),
            pltpu.VMEM((2, M, 256), jnp.bfloat16),
            pltpu.VMEM((2, M, 128), jnp.bfloat16),
            pltpu.VMEM((2, M, 128), jnp.bfloat16),
            pltpu.VMEM((2, M, 128), jnp.bfloat16),
            pltpu.VMEM((2, M, 128), jnp.bfloat16),
            pltpu.VMEM((2, M, 256), jnp.bfloat16),
            pltpu.VMEM((2, M, 256), jnp.bfloat16),
            pltpu.SemaphoreType.DMA((8,)),
            pltpu.SemaphoreType.DMA((8,)),
        ],
        compiler_params=pltpu.CompilerParams(collective_id=0),
    )(x, w_mat)


# device time: 27132 ns/iter; 1.2179x vs baseline; 1.2179x over previous
import jax
import jax.numpy as jnp
from jax import lax
from jax.experimental import pallas as pl
from jax.experimental.pallas import tpu as pltpu

M = 1024
N = 1024
R = 4
RB = M // R


def kernel(x, w_mat):
    def body(x_ref, w_ref, out_ref, s1, r1, s2, r2, s3, r3, s4, r4,
             ssem, rsem):
        p = lax.axis_index("i")
        b0 = p & 1
        b1 = p >> 1
        n1 = p ^ 1
        n2 = 3 - p

        barrier = pltpu.get_barrier_semaphore()
        pl.semaphore_signal(barrier, inc=1, device_id=(n1,),
                            device_id_type=pl.DeviceIdType.MESH)
        pl.semaphore_signal(barrier, inc=1, device_id=(n2,),
                            device_id_type=pl.DeviceIdType.MESH)
        pl.semaphore_wait(barrier, 2)

        k1 = 256 * (((p + 1) >> 1) & 1)
        k2 = 512 + 256 * b1
        o1 = k1 + 128 * b1
        o2 = k2 + 128 * b0

        SBUF = [s1, s2, s3, s4]
        RBUF = [r1, r2, r3, r4]
        WIDTH = [256, 128, 128, 256]
        SENDS = [
            [(n1, 256 - k1), (n2, 768 - 256 * b1)],
            [(n2, k1 + 128 - 128 * b1),
             (n1, k2 + 128 - 128 * b0)],
            [(n2, o1), (n1, o2)],
            [(n1, k1), (n2, k2)],
        ]
        ACC = [[(k1, 256), (k2, 256)], [(o1, 128), (o2, 128)]]
        STORE = [
            [(k1 + 128 - 128 * b1, 128), (k2 + 128 - 128 * b0, 128)],
            [(256 - k1, 256), (768 - 256 * b1, 256)],
        ]

        def start_exchange(phase, blk):
            sbuf, width = SBUF[phase], WIDTH[phase]
            rows = slice(blk * RB, (blk + 1) * RB)
            rdmas = []
            for j, (partner, col) in enumerate(SENDS[phase]):
                sbuf[blk, j, :, :] = out_ref[rows, pl.ds(col, width)].astype(
                    jnp.bfloat16)
                rdma = pltpu.make_async_remote_copy(
                    src_ref=sbuf.at[blk, j],
                    dst_ref=RBUF[phase].at[blk, j],
                    send_sem=ssem.at[phase * 2 * R + blk * 2 + j],
                    recv_sem=rsem.at[phase * 2 * R + blk * 2 + j],
                    device_id=(partner,),
                    device_id_type=pl.DeviceIdType.MESH,
                )
                rdma.start()
                rdmas.append(rdma)
            return rdmas

        def consume(phase, blk):
            rows = slice(blk * RB, (blk + 1) * RB)
            for rd in pending[blk]:
                rd.wait_recv()
            rbuf = RBUF[phase]
            if phase <= 1:
                for j, (col, w) in enumerate(ACC[phase]):
                    sl = (rows, pl.ds(col, w))
                    out_ref[sl] = out_ref[sl] + rbuf[blk, j, :, :].astype(
                        jnp.float32)
            else:
                for j, (col, w) in enumerate(STORE[phase - 2]):
                    out_ref[rows, pl.ds(col, w)] = rbuf[blk, j, :, :].astype(
                        jnp.float32)

        wb = w_ref[...].astype(jnp.bfloat16)
        done = []

        pending = []
        for blk in range(R):
            rows = slice(blk * RB, (blk + 1) * RB)
            out_ref[rows, :] = jnp.dot(
                x_ref[rows, :].astype(jnp.bfloat16), wb,
                preferred_element_type=jnp.float32)
            pending.append(start_exchange(0, blk))

        for phase in (1, 2, 3):
            nxt = []
            for blk in range(R):
                consume(phase - 1, blk)
                done.extend(pending[blk])
                nxt.append(start_exchange(phase, blk))
            pending = nxt
        for blk in range(R):
            consume(3, blk)
            done.extend(pending[blk])

        for rd in done:
            rd.wait_send()

    return pl.pallas_call(
        body,
        out_shape=jax.ShapeDtypeStruct((M, N), jnp.float32),
        in_specs=[
            pl.BlockSpec(memory_space=pltpu.VMEM),
            pl.BlockSpec(memory_space=pltpu.VMEM),
        ],
        out_specs=pl.BlockSpec(memory_space=pltpu.VMEM),
        scratch_shapes=[
            pltpu.VMEM((R, 2, RB, 256), jnp.bfloat16),
            pltpu.VMEM((R, 2, RB, 256), jnp.bfloat16),
            pltpu.VMEM((R, 2, RB, 128), jnp.bfloat16),
            pltpu.VMEM((R, 2, RB, 128), jnp.bfloat16),
            pltpu.VMEM((R, 2, RB, 128), jnp.bfloat16),
            pltpu.VMEM((R, 2, RB, 128), jnp.bfloat16),
            pltpu.VMEM((R, 2, RB, 256), jnp.bfloat16),
            pltpu.VMEM((R, 2, RB, 256), jnp.bfloat16),
            pltpu.SemaphoreType.DMA((4 * 2 * R,)),
            pltpu.SemaphoreType.DMA((4 * 2 * R,)),
        ],
        compiler_params=pltpu.CompilerParams(collective_id=0),
    )(x, w_mat)


# device time: 24684 ns/iter; 1.3387x vs baseline; 1.0992x over previous
import jax
import jax.numpy as jnp
from jax import lax
from jax.experimental import pallas as pl
from jax.experimental.pallas import tpu as pltpu

M = 1024
N = 1024
R = 4
RB = M // R


def kernel(x, w_mat):
    def body(x_ref, w_ref, out_ref, xv, wv, acc, r0, r1,
             ssem, rsem, isem, osem):
        p = lax.axis_index("i")
        b0 = p & 1
        b1 = p >> 1
        n1 = p ^ 1
        n2 = 3 - p

        xcp = pltpu.make_async_copy(x_ref, xv, isem.at[0])
        wcp = pltpu.make_async_copy(w_ref, wv, isem.at[1])
        xcp.start()
        wcp.start()
        barrier = pltpu.get_barrier_semaphore()
        pl.semaphore_signal(barrier, inc=1, device_id=(n1,),
                            device_id_type=pl.DeviceIdType.MESH)
        pl.semaphore_signal(barrier, inc=1, device_id=(n2,),
                            device_id_type=pl.DeviceIdType.MESH)
        pl.semaphore_wait(barrier, 2)
        xcp.wait()
        wcp.wait()

        k1 = 256 * (((p + 1) >> 1) & 1)
        k2 = 512 + 256 * b1
        o1 = k1 + 128 * b1
        o2 = k2 + 128 * b0

        WIDTH = [256, 128, 128, 256]
        SENDS = [
            [(n1, 256 - k1), (n2, 768 - 256 * b1)],
            [(n2, k1 + 128 - 128 * b1),
             (n1, k2 + 128 - 128 * b0)],
            [(n2, o1), (n1, o2)],
            [(n1, k1), (n2, k2)],
        ]
        RBUF = {0: r0, 1: r1}
        ACC = [[(k1, 256), (k2, 256)], [(o1, 128), (o2, 128)]]

        def start_exchange(phase, blk):
            width = WIDTH[phase]
            rows = pl.ds(blk * RB, RB)
            rdmas = []
            for j, (partner, col) in enumerate(SENDS[phase]):
                src = acc.at[rows, pl.ds(col, width)]
                if phase <= 1:
                    dst = RBUF[phase].at[blk, j]
                else:
                    dst = acc.at[rows, pl.ds(col, width)]
                rdma = pltpu.make_async_remote_copy(
                    src_ref=src,
                    dst_ref=dst,
                    send_sem=ssem.at[phase * 2 * R + blk * 2 + j],
                    recv_sem=rsem.at[phase * 2 * R + blk * 2 + j],
                    device_id=(partner,),
                    device_id_type=pl.DeviceIdType.MESH,
                )
                rdma.start()
                rdmas.append(rdma)
            return rdmas

        def consume(phase, blk):
            for rd in pending[blk]:
                rd.wait_recv()
            if phase <= 1:
                rows = pl.ds(blk * RB, RB)
                for j, (col, w) in enumerate(ACC[phase]):
                    sl = (rows, pl.ds(col, w))
                    acc[sl] = acc[sl] + RBUF[phase][blk, j, :, :]

        wb = wv[...]
        done = []

        pending = []
        for blk in range(R):
            rows = pl.ds(blk * RB, RB)
            acc[rows, :] = jnp.dot(
                xv[rows, :], wb,
                preferred_element_type=jnp.float32).astype(jnp.bfloat16)
            pending.append(start_exchange(0, blk))

        for phase in (1, 2, 3):
            nxt = []
            for blk in range(R):
                consume(phase - 1, blk)
                done.extend(pending[blk])
                nxt.append(start_exchange(phase, blk))
            pending = nxt

        ocps = []
        for blk in range(R):
            consume(3, blk)
            done.extend(pending[blk])
            rows = pl.ds(blk * RB, RB)
            ocp = pltpu.make_async_copy(
                acc.at[rows, :], out_ref.at[rows, :], osem.at[blk])
            ocp.start()
            ocps.append(ocp)

        for ocp in ocps:
            ocp.wait()
        for rd in done:
            rd.wait_send()

    return pl.pallas_call(
        body,
        out_shape=jax.ShapeDtypeStruct((M, N), jnp.bfloat16),
        in_specs=[
            pl.BlockSpec(memory_space=pl.ANY),
            pl.BlockSpec(memory_space=pl.ANY),
        ],
        out_specs=pl.BlockSpec(memory_space=pl.ANY),
        scratch_shapes=[
            pltpu.VMEM((M, 256), jnp.bfloat16),
            pltpu.VMEM((256, N), jnp.bfloat16),
            pltpu.VMEM((M, N), jnp.bfloat16),
            pltpu.VMEM((R, 2, RB, 256), jnp.bfloat16),
            pltpu.VMEM((R, 2, RB, 128), jnp.bfloat16),
            pltpu.SemaphoreType.DMA((4 * 2 * R,)),
            pltpu.SemaphoreType.DMA((4 * 2 * R,)),
            pltpu.SemaphoreType.DMA((2,)),
            pltpu.SemaphoreType.DMA((R,)),
        ],
        compiler_params=pltpu.CompilerParams(collective_id=0),
    )(x.astype(jnp.bfloat16), w_mat.astype(jnp.bfloat16))
